# baseline (speedup 1.0000x reference)
import functools

import jax
import jax.numpy as jnp
from jax import lax
from jax.experimental import pallas as pl
from jax.experimental.pallas import tpu as pltpu
from jax.experimental.pallas import tpu_sc as plsc

NUM_E = 1024
DIM = 64
CC = 0.25
BLK = 4096
NBATCH = BLK // 1024

NC = 2
NS = 16
NW = NC * NS
CHUNK = 128


def _vq_body(x_ref, w_ref, idx_ref, idx2_ref, loss_ref, acc_ref):
    i = pl.program_id(0)
    xb = jnp.transpose(x_ref[...], (0, 2, 1)).reshape(BLK, DIM)
    w = w_ref[...]
    x2 = jnp.sum(xb * xb, axis=1, keepdims=True)
    dotm = lax.dot_general(
        xb * (-2.0), w, (((1,), (1,)), ((), ())),
        preferred_element_type=jnp.float32)
    w2 = jnp.sum(w * w, axis=1)
    d = x2 + dotm + w2[None, :]
    dmin = jnp.min(d, axis=1, keepdims=True)
    ks = lax.broadcasted_iota(jnp.int32, d.shape, 1)
    idx = jnp.min(jnp.where(d == dmin, ks, NUM_E), axis=1)
    idx_ref[...] = idx
    idx2_ref[...] = idx.reshape(BLK // CHUNK, CHUNK)
    bs = jnp.sum(dmin)

    @pl.when(i == 0)
    def _():
        acc_ref[0] = 0.0

    acc_ref[0] += bs

    @pl.when(i == pl.num_programs(0) - 1)
    def _():
        loss_ref[0, 0] = acc_ref[0]


def _distance_argmin(xt, w, off, nblk):
    B = nblk * BLK
    grid = nblk
    idx, idx2d, loss = pl.pallas_call(
        _vq_body,
        grid=(grid,),
        in_specs=[
            pl.BlockSpec((NBATCH, DIM, 1024), lambda i: (i + off, 0, 0)),
            pl.BlockSpec((NUM_E, DIM), lambda i: (0, 0)),
        ],
        out_specs=[
            pl.BlockSpec((BLK,), lambda i: (i,)),
            pl.BlockSpec((BLK // CHUNK, CHUNK), lambda i: (i, 0)),
            pl.BlockSpec(memory_space=pltpu.SMEM),
        ],
        out_shape=[
            jax.ShapeDtypeStruct((B,), jnp.int32),
            jax.ShapeDtypeStruct((B // CHUNK, CHUNK), jnp.int32),
            jax.ShapeDtypeStruct((1, 1), jnp.float32),
        ],
        scratch_shapes=[pltpu.SMEM((1,), jnp.float32)],
    )(xt, w)
    return idx, idx2d, loss


def _make_sc_gather(B):
    rows_per_w = B // NW
    n_chunks = rows_per_w // CHUNK
    mesh = plsc.VectorSubcoreMesh(core_axis_name="c", subcore_axis_name="s")

    @functools.partial(
        pl.kernel,
        mesh=mesh,
        out_type=jax.ShapeDtypeStruct((B, 2 * DIM), jnp.float32),
        scratch_types=[
            pltpu.VMEM((n_chunks, CHUNK), jnp.int32),
            pltpu.VMEM((rows_per_w, 2 * DIM), jnp.float32),
            pltpu.SemaphoreType.DMA,
        ],
    )
    def sc_gather(table_hbm, idx_hbm, out_hbm, idx_v, rows_v, sem):
        wid = lax.axis_index("s") * NC + lax.axis_index("c")
        pltpu.sync_copy(idx_hbm.at[pl.ds(wid * n_chunks, n_chunks)], idx_v)
        copies = []
        for j in range(n_chunks):
            copies.append(pltpu.async_copy(
                table_hbm.at[idx_v.at[j]],
                rows_v.at[pl.ds(j * CHUNK, CHUNK)], sem))
        for c in copies:
            c.wait()
        pltpu.sync_copy(rows_v, out_hbm.at[pl.ds(wid * rows_per_w, rows_per_w)])

    return sc_gather


def kernel(inputs, embedding_weight):
    shape = inputs.shape
    xt = jnp.transpose(inputs, (0, 2, 1))
    B = shape[0] * shape[1]
    nblk = B // BLK
    na = nblk // 2
    wpad = jnp.pad(embedding_weight, ((0, 0), (0, 2 * DIM - DIM)))
    gather = _make_sc_gather(B // 2)
    idx_a, idx2d_a, loss_a = _distance_argmin(xt, embedding_weight, 0, na)
    q_a = gather(wpad, idx2d_a)
    idx_b, idx2d_b, loss_b = _distance_argmin(xt, embedding_weight, na, nblk - na)
    q_b = gather(wpad, idx2d_b)
    q = jnp.concatenate([q_a[:, :DIM], q_b[:, :DIM]], axis=0)
    idx = jnp.concatenate([idx_a, idx_b])
    loss = (loss_a[0, 0] + loss_b[0, 0]) * (CC / (B * DIM))
    return q.reshape(shape), loss, idx.reshape(shape[:-1])


# speedup vs baseline: 1.0429x; 1.0429x over previous
import functools

import jax
import jax.numpy as jnp
from jax import lax
from jax.experimental import pallas as pl
from jax.experimental.pallas import tpu as pltpu
from jax.experimental.pallas import tpu_sc as plsc

NUM_E = 1024
DIM = 64
CC = 0.25
BLK = 4096
NBATCH = BLK // 1024

NC = 2
NS = 16
NW = NC * NS
CHUNK = 128


def _vq_body(x_ref, w_ref, idx_ref, idx2_ref, loss_ref, acc_ref):
    i = pl.program_id(0)
    xb = jnp.transpose(x_ref[...], (0, 2, 1)).reshape(BLK, DIM)
    w = w_ref[...]
    x2 = jnp.sum(xb * xb, axis=1, keepdims=True)
    dotm = lax.dot_general(
        xb * (-2.0), w, (((1,), (1,)), ((), ())),
        preferred_element_type=jnp.float32)
    w2 = jnp.sum(w * w, axis=1)
    d = x2 + dotm + w2[None, :]
    dmin = jnp.min(d, axis=1, keepdims=True)
    ks = lax.broadcasted_iota(jnp.int32, d.shape, 1)
    idx = jnp.min(jnp.where(d == dmin, ks, NUM_E), axis=1)
    idx_ref[...] = idx
    idx2_ref[...] = idx.reshape(BLK // CHUNK, CHUNK)
    bs = jnp.sum(dmin)

    @pl.when(i == 0)
    def _():
        acc_ref[0] = 0.0

    acc_ref[0] += bs

    @pl.when(i == pl.num_programs(0) - 1)
    def _():
        loss_ref[0, 0] = acc_ref[0]


def _distance_argmin(xt, w, off, nblk):
    B = nblk * BLK
    grid = nblk
    idx, idx2d, loss = pl.pallas_call(
        _vq_body,
        grid=(grid,),
        in_specs=[
            pl.BlockSpec((NBATCH, DIM, 1024), lambda i: (i + off, 0, 0)),
            pl.BlockSpec((NUM_E, DIM), lambda i: (0, 0)),
        ],
        out_specs=[
            pl.BlockSpec((BLK,), lambda i: (i,)),
            pl.BlockSpec((BLK // CHUNK, CHUNK), lambda i: (i, 0)),
            pl.BlockSpec(memory_space=pltpu.SMEM),
        ],
        out_shape=[
            jax.ShapeDtypeStruct((B,), jnp.int32),
            jax.ShapeDtypeStruct((B // CHUNK, CHUNK), jnp.int32),
            jax.ShapeDtypeStruct((1, 1), jnp.float32),
        ],
        scratch_shapes=[pltpu.SMEM((1,), jnp.float32)],
    )(xt, w)
    return idx, idx2d, loss


def _make_sc_gather(B):
    rows_per_w = B // NW
    n_chunks = rows_per_w // CHUNK
    mesh = plsc.VectorSubcoreMesh(core_axis_name="c", subcore_axis_name="s")

    @functools.partial(
        pl.kernel,
        mesh=mesh,
        out_type=jax.ShapeDtypeStruct((B, 2 * DIM), jnp.float32),
        scratch_types=[
            pltpu.VMEM((n_chunks, CHUNK), jnp.int32),
            pltpu.VMEM((rows_per_w, 2 * DIM), jnp.float32),
            pltpu.SemaphoreType.DMA,
        ],
    )
    def sc_gather(table_hbm, idx_hbm, out_hbm, idx_v, rows_v, sem):
        wid = lax.axis_index("s") * NC + lax.axis_index("c")
        pltpu.sync_copy(idx_hbm.at[pl.ds(wid * n_chunks, n_chunks)], idx_v)
        copies = []
        for j in range(n_chunks):
            copies.append(pltpu.async_copy(
                table_hbm.at[idx_v.at[j]],
                rows_v.at[pl.ds(j * CHUNK, CHUNK)], sem))
        for c in copies:
            c.wait()
        pltpu.sync_copy(rows_v, out_hbm.at[pl.ds(wid * rows_per_w, rows_per_w)])

    return sc_gather


def kernel(inputs, embedding_weight):
    shape = inputs.shape
    xt = jnp.transpose(inputs, (0, 2, 1))
    B = shape[0] * shape[1]
    idx, idx2d, loss = _distance_argmin(xt, embedding_weight, 0, B // BLK)
    wpad = jnp.pad(embedding_weight, ((0, 0), (0, 2 * DIM - DIM)))
    q = _make_sc_gather(B)(wpad, idx2d)[:, :DIM]
    loss = loss[0, 0] * (CC / (B * DIM))
    return q.reshape(shape), loss, idx.reshape(shape[:-1])
